# Initial kernel scaffold; baseline (speedup 1.0000x reference)
import jax
import jax.numpy as jnp
from jax.experimental import pallas as pl


def kernel(head_index, tail_index, head_neg_index, tail_neg_index, rel_index, emb):
    raise NotImplementedError("write your pallas kernel here")


# speedup vs baseline: 2.6855x; 2.6855x over previous
import functools

import jax
import jax.numpy as jnp
from jax import lax
from jax.experimental import pallas as pl
from jax.experimental.pallas import tpu as pltpu
from jax.experimental.pallas import tpu_sc as plsc

ENT_SIZE = 1000000
DIM = 32
NUM_CHUNK = 16
POS_NUM = 1024
NEG_NUM = 1024
PN = POS_NUM + NEG_NUM

_IDX_ROWS = 4 * NUM_CHUNK * POS_NUM // 128
_NW = 32
_ROWS_PER_W = _IDX_ROWS // _NW


@functools.cache
def _make_sc_gather():
    @functools.partial(
        pl.kernel,
        mesh=plsc.VectorSubcoreMesh(core_axis_name="c", subcore_axis_name="s"),
        out_type=jax.ShapeDtypeStruct((_IDX_ROWS, 128, DIM), jnp.float32),
        scratch_types=[
            pltpu.VMEM((_ROWS_PER_W, 128), jnp.int32),
            pltpu.VMEM((_ROWS_PER_W, 128, DIM), jnp.float32),
            pltpu.SemaphoreType.DMA,
        ],
        compiler_params=pltpu.CompilerParams(use_tc_tiling_on_sc=False),
    )
    def _sc_gather(emb_hbm, idx_hbm, out_hbm, idx_v, rows_v, sem):
        wid = lax.axis_index("s") * 2 + lax.axis_index("c")
        base = wid * _ROWS_PER_W
        pltpu.sync_copy(idx_hbm.at[pl.ds(base, _ROWS_PER_W)], idx_v)
        copies = [
            pltpu.async_copy(emb_hbm.at[idx_v.at[j]], rows_v.at[j], sem)
            for j in range(_ROWS_PER_W)
        ]
        for cp in copies:
            cp.wait()
        pltpu.sync_copy(rows_v, out_hbm.at[pl.ds(base, _ROWS_PER_W)])

    return _sc_gather


_TP = 256


def _tc_score_body(g_ref, pos_ref, hn_ref, tn_ref):
    pid = pl.program_id(1)
    ht = g_ref[0, 0, pl.ds(pid * _TP, _TP), :]
    tt = g_ref[1, 0, pl.ds(pid * _TP, _TP), :]
    head_full = g_ref[0, 0]
    tail_full = g_ref[1, 0]
    hne = g_ref[2, 0]
    tne = g_ref[3, 0]
    dn = (((1,), (1,)), ((), ()))
    s_ht = lax.dot_general(ht, tail_full, dn, preferred_element_type=jnp.float32)
    s_hn = lax.dot_general(ht, tne, dn, preferred_element_type=jnp.float32)
    s_th = lax.dot_general(tt, head_full, dn, preferred_element_type=jnp.float32)
    s_tn = lax.dot_general(tt, hne, dn, preferred_element_type=jnp.float32)
    rows = pid * _TP + lax.broadcasted_iota(jnp.int32, (_TP, POS_NUM), 0)
    cols = lax.broadcasted_iota(jnp.int32, (_TP, POS_NUM), 1)
    neg = jnp.where(rows == cols, jnp.float32(-1000000000.0), jnp.float32(0.0))
    hn_ref[0, :, 0:POS_NUM] = s_ht + neg
    hn_ref[0, :, POS_NUM:PN] = s_hn
    tn_ref[0, :, 0:POS_NUM] = s_th + neg
    tn_ref[0, :, POS_NUM:PN] = s_tn
    pos_ref[0] = jnp.sum(ht * tt, axis=1, keepdims=True)


_tc_score = pl.pallas_call(
    _tc_score_body,
    grid=(NUM_CHUNK, POS_NUM // _TP),
    in_specs=[
        pl.BlockSpec((4, 1, POS_NUM, DIM), lambda c, p: (0, c, 0, 0)),
    ],
    out_specs=[
        pl.BlockSpec((1, _TP, 1), lambda c, p: (c, p, 0)),
        pl.BlockSpec((1, _TP, PN), lambda c, p: (c, p, 0)),
        pl.BlockSpec((1, _TP, PN), lambda c, p: (c, p, 0)),
    ],
    out_shape=[
        jax.ShapeDtypeStruct((NUM_CHUNK, POS_NUM, 1), jnp.float32),
        jax.ShapeDtypeStruct((NUM_CHUNK, POS_NUM, PN), jnp.float32),
        jax.ShapeDtypeStruct((NUM_CHUNK, POS_NUM, PN), jnp.float32),
    ],
)


def kernel(head_index, tail_index, head_neg_index, tail_neg_index, rel_index, emb):
    del rel_index
    idx = jnp.stack(
        [head_index, tail_index, head_neg_index, tail_neg_index]
    ).astype(jnp.int32)
    idx_grid = idx.reshape(_IDX_ROWS, 128)
    gathered = _make_sc_gather()(emb, idx_grid)
    g = gathered.reshape(4, NUM_CHUNK, POS_NUM, DIM)
    pos, hn, tn = _tc_score(g)
    pos2 = pos.reshape(NUM_CHUNK * POS_NUM, 1)
    return (
        pos2,
        pos2,
        hn.reshape(NUM_CHUNK * POS_NUM, PN),
        tn.reshape(NUM_CHUNK * POS_NUM, PN),
    )
